# Initial kernel scaffold; baseline (speedup 1.0000x reference)
import jax
import jax.numpy as jnp
from jax.experimental import pallas as pl


def kernel(x, edge_index, y, emb, W_iou, U_iou, b_iou, W_f, U_f, b_f, W_lin, b_lin):
    raise NotImplementedError("write your pallas kernel here")


# speedup vs baseline: 2.1623x; 2.1623x over previous
import functools

import jax
import jax.numpy as jnp
from jax import lax
from jax.experimental import pallas as pl
from jax.experimental.pallas import tpu as pltpu
from jax.experimental.pallas import tpu_sc as plsc

N = 10000
E = 160000
X = 256
H = 256
HH = 128
C = 5

NW = 32
NP = 10240
GCH = 80
EPS = E // 16
EC = 80
NCH = EPS // EC
RPS = 624
RB = 1000
GRID = N // RB


def _sc_mesh():
    return plsc.VectorSubcoreMesh(core_axis_name="c", subcore_axis_name="s")


def _emb_gather_body(emb_hbm, xpad_hbm, out_hbm, idx_v, rows_v, sem):
    cid = lax.axis_index("c")
    sid = lax.axis_index("s")
    wid = sid * 2 + cid
    base = wid * (NP // NW)
    for k in range(NP // NW // GCH):
        off = base + k * GCH
        pltpu.sync_copy(xpad_hbm.at[pl.ds(off, GCH)], idx_v)
        pltpu.async_copy(emb_hbm.at[idx_v], rows_v, sem).wait()
        pltpu.sync_copy(rows_v, out_hbm.at[pl.ds(off, GCH)])


def _emb_gather(emb, x_pad):
    return pl.kernel(
        _emb_gather_body,
        out_type=jax.ShapeDtypeStruct((NP, X), jnp.float32),
        mesh=_sc_mesh(),
        scratch_types=[
            pltpu.VMEM((GCH,), jnp.int32),
            pltpu.VMEM((GCH, X), jnp.float32),
            pltpu.SemaphoreType.DMA,
        ],
    )(emb, x_pad)


def _edge_body(h_hbm, c_hbm, hf_hbm, xf_hbm, srcp_hbm, dstp_hbm, dst_hbm,
               zeros_hbm, ht_hbm, fc_hbm,
               acc, srcv, dstv, rows_hf, rows_c, rows_xf, prod, sem):
    cid = lax.axis_index("c")
    sid = lax.axis_index("s")
    rbase = sid * RPS
    ebase = sid * EPS
    eoff = cid * E
    noff = cid * N
    tail = 16 * RPS

    def _zero_acc():
        pltpu.sync_copy(zeros_hbm.at[pl.ds(rbase, RPS)],
                        acc.at[pl.ds(rbase, RPS)])

        @pl.when(sid == 15)
        def _():
            pltpu.sync_copy(zeros_hbm.at[pl.ds(tail, N - tail)],
                            acc.at[pl.ds(tail, N - tail)])

    def _flush_acc(out_hbm):
        pltpu.sync_copy(acc.at[pl.ds(rbase, RPS)],
                        out_hbm.at[pl.ds(noff + rbase, RPS)])

        @pl.when(sid == 15)
        def _():
            pltpu.sync_copy(acc.at[pl.ds(tail, N - tail)],
                            out_hbm.at[pl.ds(noff + tail, N - tail)])

    _zero_acc()
    plsc.subcore_barrier()

    def pass_a(k, carry):
        off = ebase + k * EC
        pltpu.sync_copy(srcp_hbm.at[pl.ds(eoff + off, EC)], srcv)
        pltpu.sync_copy(dst_hbm.at[pl.ds(off, EC)], dstv)
        pltpu.async_copy(h_hbm.at[srcv], prod, sem).wait()
        pltpu.sync_copy(prod, acc.at[dstv], add=True)
        return carry

    lax.fori_loop(0, NCH, pass_a, 0)
    plsc.subcore_barrier()
    _flush_acc(ht_hbm)
    _zero_acc()
    plsc.subcore_barrier()

    def pass_b(k, carry):
        off = ebase + k * EC
        pltpu.sync_copy(srcp_hbm.at[pl.ds(eoff + off, EC)], srcv)
        pltpu.async_copy(hf_hbm.at[srcv], rows_hf, sem).wait()
        pltpu.async_copy(c_hbm.at[srcv], rows_c, sem).wait()
        pltpu.sync_copy(dstp_hbm.at[pl.ds(eoff + off, EC)], srcv)
        pltpu.async_copy(xf_hbm.at[srcv], rows_xf, sem).wait()
        pltpu.sync_copy(dst_hbm.at[pl.ds(off, EC)], dstv)

        def row(r, rc):
            for j in range(HH // 16):
                sl = pl.ds(j * 16, 16)
                z = rows_hf[r, sl] + rows_xf[r, sl]
                f = 1.0 / (1.0 + jnp.exp(-z))
                prod[r, sl] = f * rows_c[r, sl]
            return rc

        lax.fori_loop(0, EC, row, 0)
        pltpu.sync_copy(prod, acc.at[dstv], add=True)
        return carry

    lax.fori_loop(0, NCH, pass_b, 0)
    plsc.subcore_barrier()
    _flush_acc(fc_hbm)


def _edge_phase(h2, c2, hf2, xf2, src_plus, dst_plus, dst, zeros_half):
    return pl.kernel(
        _edge_body,
        out_type=(jax.ShapeDtypeStruct((2 * N, HH), jnp.float32),
                  jax.ShapeDtypeStruct((2 * N, HH), jnp.float32)),
        mesh=_sc_mesh(),
        scratch_types=[
            pltpu.VMEM_SHARED((N, HH), jnp.float32),
            pltpu.VMEM((EC,), jnp.int32),
            pltpu.VMEM((EC,), jnp.int32),
            pltpu.VMEM((EC, HH), jnp.float32),
            pltpu.VMEM((EC, HH), jnp.float32),
            pltpu.VMEM((EC, HH), jnp.float32),
            pltpu.VMEM((EC, HH), jnp.float32),
            pltpu.SemaphoreType.DMA,
        ],
    )(h2, c2, hf2, xf2, src_plus, dst_plus, dst, zeros_half)


def _split_store(ref, full):
    ref[0] = full[:, :HH]
    ref[1] = full[:, HH:]


def _pre_body(xe_ref, wiou_ref, biou_ref, wf_ref, bf_ref, uf_ref,
              xiou_ref, xf_ref, h0_ref, c0_ref, hf1_ref):
    xe = xe_ref[...]
    xiou = jnp.dot(xe, wiou_ref[...]) + biou_ref[...]
    xiou_ref[...] = xiou
    xf = jnp.dot(xe, wf_ref[...]) + bf_ref[...]
    _split_store(xf_ref, xf)
    i = jax.nn.sigmoid(xiou[:, :H])
    o = jax.nn.sigmoid(xiou[:, H:2 * H])
    u = jnp.tanh(xiou[:, 2 * H:])
    c0 = i * u
    h0 = o * jnp.tanh(c0)
    _split_store(c0_ref, c0)
    _split_store(h0_ref, h0)
    _split_store(hf1_ref, jnp.dot(h0, uf_ref[...]))


def _tc_pre(xe_pad, W_iou, b_iou, W_f, b_f, U_f):
    out3 = jax.ShapeDtypeStruct((2, N, HH), jnp.float32)
    full = lambda *_: (0, 0)
    split3 = pl.BlockSpec((2, RB, HH), lambda i: (0, i, 0))
    return pl.pallas_call(
        _pre_body,
        grid=(GRID,),
        in_specs=[
            pl.BlockSpec((RB, X), lambda i: (i, 0)),
            pl.BlockSpec((X, 3 * H), full),
            pl.BlockSpec((1, 3 * H), full),
            pl.BlockSpec((X, H), full),
            pl.BlockSpec((1, H), full),
            pl.BlockSpec((H, H), full),
        ],
        out_specs=[
            pl.BlockSpec((RB, 3 * H), lambda i: (i, 0)),
            split3, split3, split3, split3,
        ],
        out_shape=[jax.ShapeDtypeStruct((N, 3 * H), jnp.float32),
                   out3, out3, out3, out3],
    )(xe_pad, W_iou, b_iou, W_f, b_f, U_f)


def _gate_update(ht_ref, fc_ref, xiou_ref, uiou_ref):
    htil = jnp.concatenate([ht_ref[0], ht_ref[1]], axis=1)
    iou = xiou_ref[...] + jnp.dot(htil, uiou_ref[...])
    i = jax.nn.sigmoid(iou[:, :H])
    o = jax.nn.sigmoid(iou[:, H:2 * H])
    u = jnp.tanh(iou[:, 2 * H:])
    c = i * u + jnp.concatenate([fc_ref[0], fc_ref[1]], axis=1)
    h = o * jnp.tanh(c)
    return h, c


def _step_body(ht_ref, fc_ref, xiou_ref, uiou_ref, uf_ref,
               h_ref, c_ref, hf_ref):
    h, c = _gate_update(ht_ref, fc_ref, xiou_ref, uiou_ref)
    _split_store(h_ref, h)
    _split_store(c_ref, c)
    _split_store(hf_ref, jnp.dot(h, uf_ref[...]))


def _tc_step(ht2, fc2, xiou, U_iou, U_f):
    out3 = jax.ShapeDtypeStruct((2, N, HH), jnp.float32)
    full = lambda *_: (0, 0)
    split3 = pl.BlockSpec((2, RB, HH), lambda i: (0, i, 0))
    return pl.pallas_call(
        _step_body,
        grid=(GRID,),
        in_specs=[
            split3, split3,
            pl.BlockSpec((RB, 3 * H), lambda i: (i, 0)),
            pl.BlockSpec((H, 3 * H), full),
            pl.BlockSpec((H, H), full),
        ],
        out_specs=[split3, split3, split3],
        out_shape=[out3, out3, out3],
    )(ht2, fc2, xiou, U_iou, U_f)


def _final_body(ht_ref, fc_ref, xiou_ref, y_ref, uiou_ref, wlin_ref, blin_ref,
                loss_ref):
    h, _ = _gate_update(ht_ref, fc_ref, xiou_ref, uiou_ref)
    logits = jnp.dot(h, wlin_ref[...]) + blin_ref[...]
    m = jnp.max(logits, axis=1, keepdims=True)
    lse = m + jnp.log(jnp.sum(jnp.exp(logits - m), axis=1, keepdims=True))
    y = y_ref[...]
    sel = lax.broadcasted_iota(jnp.int32, (RB, 128), 1) == y
    ly = jnp.sum(jnp.where(sel, logits, 0.0), axis=1, keepdims=True)
    part = jnp.sum(lse - ly, keepdims=True).reshape(1, 1) * (1.0 / N)

    @pl.when(pl.program_id(0) == 0)
    def _():
        loss_ref[...] = jnp.zeros((1, 1), jnp.float32)

    loss_ref[...] += part


def _tc_final(ht2, fc2, xiou, y2d, U_iou, W_lin_pad, b_lin_pad):
    full = lambda *_: (0, 0)
    split3 = pl.BlockSpec((2, RB, HH), lambda i: (0, i, 0))
    return pl.pallas_call(
        _final_body,
        grid=(GRID,),
        in_specs=[
            split3, split3,
            pl.BlockSpec((RB, 3 * H), lambda i: (i, 0)),
            pl.BlockSpec((RB, 1), lambda i: (i, 0)),
            pl.BlockSpec((H, 3 * H), full),
            pl.BlockSpec((H, 128), full),
            pl.BlockSpec((1, 128), full),
        ],
        out_specs=pl.BlockSpec((1, 1), full),
        out_shape=jax.ShapeDtypeStruct((1, 1), jnp.float32),
    )(ht2, fc2, xiou, y2d, U_iou, W_lin_pad, b_lin_pad)


def kernel(x, edge_index, y, emb, W_iou, U_iou, b_iou, W_f, U_f, b_f,
           W_lin, b_lin):
    src = edge_index[0].astype(jnp.int32)
    dst = edge_index[1].astype(jnp.int32)
    src_plus = jnp.concatenate([src, src + N])
    dst_plus = jnp.concatenate([dst, dst + N])
    x_pad = jnp.concatenate([x.astype(jnp.int32),
                             jnp.zeros((NP - N,), jnp.int32)])
    zeros_half = jnp.zeros((N, HH), jnp.float32)
    W_lin_pad = jnp.concatenate(
        [W_lin, jnp.zeros((H, 128 - C), jnp.float32)], axis=1)
    b_lin_pad = jnp.concatenate(
        [b_lin, jnp.full((128 - C,), -1e30, jnp.float32)]).reshape(1, 128)
    y2d = y.astype(jnp.int32).reshape(N, 1)

    xe_pad = _emb_gather(emb, x_pad)
    xiou, xf2, h2, c2, hf2 = _tc_pre(
        xe_pad, W_iou, b_iou.reshape(1, -1), W_f, b_f.reshape(1, -1), U_f)

    xf_flat = xf2.reshape(2 * N, HH)
    for _ in range(2):
        ht, fc = _edge_phase(h2.reshape(2 * N, HH), c2.reshape(2 * N, HH),
                             hf2.reshape(2 * N, HH), xf_flat,
                             src_plus, dst_plus, dst, zeros_half)
        h2, c2, hf2 = _tc_step(ht.reshape(2, N, HH), fc.reshape(2, N, HH),
                               xiou, U_iou, U_f)

    ht, fc = _edge_phase(h2.reshape(2 * N, HH), c2.reshape(2 * N, HH),
                         hf2.reshape(2 * N, HH), xf_flat,
                         src_plus, dst_plus, dst, zeros_half)
    loss = _tc_final(ht.reshape(2, N, HH), fc.reshape(2, N, HH),
                     xiou, y2d, U_iou, W_lin_pad, b_lin_pad)
    return loss[0, 0]


# speedup vs baseline: 3.2717x; 1.5130x over previous
import functools

import jax
import jax.numpy as jnp
from jax import lax
from jax.experimental import pallas as pl
from jax.experimental.pallas import tpu as pltpu
from jax.experimental.pallas import tpu_sc as plsc

N = 10000
E = 160000
X = 256
H = 256
HH = 128
C = 5

NW = 32
NP = 10240
GCH = 80
EPS = E // 16
EC = 40
NCH = EPS // EC
NP2 = NCH // 2
RPS = 624
RB = 1000
GRID = N // RB


def _sc_mesh():
    return plsc.VectorSubcoreMesh(core_axis_name="c", subcore_axis_name="s")


def _emb_gather_body(emb_hbm, xpad_hbm, out_hbm, idx_v, rows_v, sem):
    cid = lax.axis_index("c")
    sid = lax.axis_index("s")
    wid = sid * 2 + cid
    base = wid * (NP // NW)
    for k in range(NP // NW // GCH):
        off = base + k * GCH
        pltpu.sync_copy(xpad_hbm.at[pl.ds(off, GCH)], idx_v)
        pltpu.async_copy(emb_hbm.at[idx_v], rows_v, sem).wait()
        pltpu.sync_copy(rows_v, out_hbm.at[pl.ds(off, GCH)])


def _emb_gather(emb, x_pad):
    return pl.kernel(
        _emb_gather_body,
        out_type=jax.ShapeDtypeStruct((NP, X), jnp.float32),
        mesh=_sc_mesh(),
        scratch_types=[
            pltpu.VMEM((GCH,), jnp.int32),
            pltpu.VMEM((GCH, X), jnp.float32),
            pltpu.SemaphoreType.DMA,
        ],
    )(emb, x_pad)


def _edge_body(h_hbm, c_hbm, hf_hbm, xf_hbm, srcp_hbm, dstp_hbm, dst_hbm,
               zeros_hbm, ht_hbm, fc_hbm,
               acc, srcv0, srcv1, dpv0, dpv1, dstv0, dstv1,
               hf0, hf1, cb0, cb1, xfb0, xfb1, p0, p1,
               sg0, sg1, ss0, ss1):
    cid = lax.axis_index("c")
    sid = lax.axis_index("s")
    ebase = sid * EPS
    eoff = cid * E
    noff = cid * N
    rbase = sid * RPS
    tail = 16 * RPS

    def _zero_acc():
        pltpu.sync_copy(zeros_hbm.at[pl.ds(rbase, RPS)],
                        acc.at[pl.ds(rbase, RPS)])

        @pl.when(sid == 15)
        def _():
            pltpu.sync_copy(zeros_hbm.at[pl.ds(tail, N - tail)],
                            acc.at[pl.ds(tail, N - tail)])

    def _flush_acc(out_hbm):
        pltpu.sync_copy(acc.at[pl.ds(rbase, RPS)],
                        out_hbm.at[pl.ds(noff + rbase, RPS)])

        @pl.when(sid == 15)
        def _():
            pltpu.sync_copy(acc.at[pl.ds(tail, N - tail)],
                            out_hbm.at[pl.ds(noff + tail, N - tail)])

    def _wait_scat(buf, dstv, sem):
        pltpu.make_async_copy(buf, acc.at[dstv], sem).wait()

    _zero_acc()
    plsc.subcore_barrier()

    def _loadA(k, srcv, dstv):
        off = ebase + k * EC
        pltpu.sync_copy(srcp_hbm.at[pl.ds(eoff + off, EC)], srcv)
        pltpu.sync_copy(dst_hbm.at[pl.ds(off, EC)], dstv)

    _loadA(0, srcv0, dstv0)
    pltpu.async_copy(h_hbm.at[srcv0], p0, sg0)

    def pass_a(t, carry):
        o = 2 * t + 1

        @pl.when(t > 0)
        def _():
            _wait_scat(p1, dstv1, ss1)

        _loadA(o, srcv1, dstv1)
        pltpu.async_copy(h_hbm.at[srcv1], p1, sg1)

        pltpu.make_async_copy(h_hbm.at[srcv0], p0, sg0).wait()
        pltpu.async_copy(p0, acc.at[dstv0], ss0, add=True)

        @pl.when(t < NP2 - 1)
        def _():
            _wait_scat(p0, dstv0, ss0)
            _loadA(o + 1, srcv0, dstv0)
            pltpu.async_copy(h_hbm.at[srcv0], p0, sg0)

        pltpu.make_async_copy(h_hbm.at[srcv1], p1, sg1).wait()
        pltpu.async_copy(p1, acc.at[dstv1], ss1, add=True)
        return carry

    lax.fori_loop(0, NP2, pass_a, 0)
    _wait_scat(p0, dstv0, ss0)
    _wait_scat(p1, dstv1, ss1)
    plsc.subcore_barrier()
    _flush_acc(ht_hbm)
    _zero_acc()
    plsc.subcore_barrier()

    def _loadB(k, srcv, dpv, dstv):
        off = ebase + k * EC
        pltpu.sync_copy(srcp_hbm.at[pl.ds(eoff + off, EC)], srcv)
        pltpu.sync_copy(dstp_hbm.at[pl.ds(eoff + off, EC)], dpv)
        pltpu.sync_copy(dst_hbm.at[pl.ds(off, EC)], dstv)

    def _fireB(srcv, dpv, hfb, cb, xfb, sg):
        pltpu.async_copy(hf_hbm.at[srcv], hfb, sg)
        pltpu.async_copy(c_hbm.at[srcv], cb, sg)
        pltpu.async_copy(xf_hbm.at[dpv], xfb, sg)

    def _waitB(srcv, dpv, hfb, cb, xfb, sg):
        pltpu.make_async_copy(hf_hbm.at[srcv], hfb, sg).wait()
        pltpu.make_async_copy(c_hbm.at[srcv], cb, sg).wait()
        pltpu.make_async_copy(xf_hbm.at[dpv], xfb, sg).wait()

    def _compute(hfb, cb, xfb, prod):
        @functools.partial(plsc.parallel_loop, 0, EC, unroll=2)
        def _(r):
            for j in range(HH // 16):
                sl = pl.ds(j * 16, 16)
                z = hfb[r, sl] + xfb[r, sl]
                prod[r, sl] = cb[r, sl] / (1.0 + jnp.exp(-z))

    _loadB(0, srcv0, dpv0, dstv0)
    _fireB(srcv0, dpv0, hf0, cb0, xfb0, sg0)

    def pass_b(t, carry):
        o = 2 * t + 1

        @pl.when(t > 0)
        def _():
            _wait_scat(p1, dstv1, ss1)

        _loadB(o, srcv1, dpv1, dstv1)
        _fireB(srcv1, dpv1, hf1, cb1, xfb1, sg1)

        _waitB(srcv0, dpv0, hf0, cb0, xfb0, sg0)
        _compute(hf0, cb0, xfb0, p0)
        pltpu.async_copy(p0, acc.at[dstv0], ss0, add=True)

        @pl.when(t < NP2 - 1)
        def _():
            _wait_scat(p0, dstv0, ss0)
            _loadB(o + 1, srcv0, dpv0, dstv0)
            _fireB(srcv0, dpv0, hf0, cb0, xfb0, sg0)

        _waitB(srcv1, dpv1, hf1, cb1, xfb1, sg1)
        _compute(hf1, cb1, xfb1, p1)
        pltpu.async_copy(p1, acc.at[dstv1], ss1, add=True)
        return carry

    lax.fori_loop(0, NP2, pass_b, 0)
    _wait_scat(p0, dstv0, ss0)
    _wait_scat(p1, dstv1, ss1)
    plsc.subcore_barrier()
    _flush_acc(fc_hbm)


def _edge_phase(h2, c2, hf2, xf2, src_plus, dst_plus, dst, zeros_half):
    idx = lambda: pltpu.VMEM((EC,), jnp.int32)
    buf = lambda: pltpu.VMEM((EC, HH), jnp.float32)
    return pl.kernel(
        _edge_body,
        out_type=(jax.ShapeDtypeStruct((2 * N, HH), jnp.float32),
                  jax.ShapeDtypeStruct((2 * N, HH), jnp.float32)),
        mesh=_sc_mesh(),
        scratch_types=[
            pltpu.VMEM_SHARED((N, HH), jnp.float32),
            idx(), idx(), idx(), idx(), idx(), idx(),
            buf(), buf(), buf(), buf(), buf(), buf(), buf(), buf(),
            pltpu.SemaphoreType.DMA, pltpu.SemaphoreType.DMA,
            pltpu.SemaphoreType.DMA, pltpu.SemaphoreType.DMA,
        ],
    )(h2, c2, hf2, xf2, src_plus, dst_plus, dst, zeros_half)


def _split_store(ref, full):
    ref[0] = full[:, :HH]
    ref[1] = full[:, HH:]


def _pre_body(xe_ref, wiou_ref, biou_ref, wf_ref, bf_ref, uf_ref,
              xiou_ref, xf_ref, h0_ref, c0_ref, hf1_ref):
    xe = xe_ref[...]
    xiou = jnp.dot(xe, wiou_ref[...]) + biou_ref[...]
    xiou_ref[...] = xiou
    xf = jnp.dot(xe, wf_ref[...]) + bf_ref[...]
    _split_store(xf_ref, xf)
    i = jax.nn.sigmoid(xiou[:, :H])
    o = jax.nn.sigmoid(xiou[:, H:2 * H])
    u = jnp.tanh(xiou[:, 2 * H:])
    c0 = i * u
    h0 = o * jnp.tanh(c0)
    _split_store(c0_ref, c0)
    _split_store(h0_ref, h0)
    _split_store(hf1_ref, jnp.dot(h0, uf_ref[...]))


def _tc_pre(xe_pad, W_iou, b_iou, W_f, b_f, U_f):
    out3 = jax.ShapeDtypeStruct((2, N, HH), jnp.float32)
    full = lambda *_: (0, 0)
    split3 = pl.BlockSpec((2, RB, HH), lambda i: (0, i, 0))
    return pl.pallas_call(
        _pre_body,
        grid=(GRID,),
        in_specs=[
            pl.BlockSpec((RB, X), lambda i: (i, 0)),
            pl.BlockSpec((X, 3 * H), full),
            pl.BlockSpec((1, 3 * H), full),
            pl.BlockSpec((X, H), full),
            pl.BlockSpec((1, H), full),
            pl.BlockSpec((H, H), full),
        ],
        out_specs=[
            pl.BlockSpec((RB, 3 * H), lambda i: (i, 0)),
            split3, split3, split3, split3,
        ],
        out_shape=[jax.ShapeDtypeStruct((N, 3 * H), jnp.float32),
                   out3, out3, out3, out3],
    )(xe_pad, W_iou, b_iou, W_f, b_f, U_f)


def _gate_update(ht_ref, fc_ref, xiou_ref, uiou_ref):
    htil = jnp.concatenate([ht_ref[0], ht_ref[1]], axis=1)
    iou = xiou_ref[...] + jnp.dot(htil, uiou_ref[...])
    i = jax.nn.sigmoid(iou[:, :H])
    o = jax.nn.sigmoid(iou[:, H:2 * H])
    u = jnp.tanh(iou[:, 2 * H:])
    c = i * u + jnp.concatenate([fc_ref[0], fc_ref[1]], axis=1)
    h = o * jnp.tanh(c)
    return h, c


def _step_body(ht_ref, fc_ref, xiou_ref, uiou_ref, uf_ref,
               h_ref, c_ref, hf_ref):
    h, c = _gate_update(ht_ref, fc_ref, xiou_ref, uiou_ref)
    _split_store(h_ref, h)
    _split_store(c_ref, c)
    _split_store(hf_ref, jnp.dot(h, uf_ref[...]))


def _tc_step(ht2, fc2, xiou, U_iou, U_f):
    out3 = jax.ShapeDtypeStruct((2, N, HH), jnp.float32)
    full = lambda *_: (0, 0)
    split3 = pl.BlockSpec((2, RB, HH), lambda i: (0, i, 0))
    return pl.pallas_call(
        _step_body,
        grid=(GRID,),
        in_specs=[
            split3, split3,
            pl.BlockSpec((RB, 3 * H), lambda i: (i, 0)),
            pl.BlockSpec((H, 3 * H), full),
            pl.BlockSpec((H, H), full),
        ],
        out_specs=[split3, split3, split3],
        out_shape=[out3, out3, out3],
    )(ht2, fc2, xiou, U_iou, U_f)


def _final_body(ht_ref, fc_ref, xiou_ref, y_ref, uiou_ref, wlin_ref, blin_ref,
                loss_ref):
    h, _ = _gate_update(ht_ref, fc_ref, xiou_ref, uiou_ref)
    logits = jnp.dot(h, wlin_ref[...]) + blin_ref[...]
    m = jnp.max(logits, axis=1, keepdims=True)
    lse = m + jnp.log(jnp.sum(jnp.exp(logits - m), axis=1, keepdims=True))
    y = y_ref[...]
    sel = lax.broadcasted_iota(jnp.int32, (RB, 128), 1) == y
    ly = jnp.sum(jnp.where(sel, logits, 0.0), axis=1, keepdims=True)
    part = jnp.sum(lse - ly, keepdims=True).reshape(1, 1) * (1.0 / N)

    @pl.when(pl.program_id(0) == 0)
    def _():
        loss_ref[...] = jnp.zeros((1, 1), jnp.float32)

    loss_ref[...] += part


def _tc_final(ht2, fc2, xiou, y2d, U_iou, W_lin_pad, b_lin_pad):
    full = lambda *_: (0, 0)
    split3 = pl.BlockSpec((2, RB, HH), lambda i: (0, i, 0))
    return pl.pallas_call(
        _final_body,
        grid=(GRID,),
        in_specs=[
            split3, split3,
            pl.BlockSpec((RB, 3 * H), lambda i: (i, 0)),
            pl.BlockSpec((RB, 1), lambda i: (i, 0)),
            pl.BlockSpec((H, 3 * H), full),
            pl.BlockSpec((H, 128), full),
            pl.BlockSpec((1, 128), full),
        ],
        out_specs=pl.BlockSpec((1, 1), full),
        out_shape=jax.ShapeDtypeStruct((1, 1), jnp.float32),
    )(ht2, fc2, xiou, y2d, U_iou, W_lin_pad, b_lin_pad)


def kernel(x, edge_index, y, emb, W_iou, U_iou, b_iou, W_f, U_f, b_f,
           W_lin, b_lin):
    src = edge_index[0].astype(jnp.int32)
    dst = edge_index[1].astype(jnp.int32)
    src_plus = jnp.concatenate([src, src + N])
    dst_plus = jnp.concatenate([dst, dst + N])
    x_pad = jnp.concatenate([x.astype(jnp.int32),
                             jnp.zeros((NP - N,), jnp.int32)])
    zeros_half = jnp.zeros((N, HH), jnp.float32)
    W_lin_pad = jnp.concatenate(
        [W_lin, jnp.zeros((H, 128 - C), jnp.float32)], axis=1)
    b_lin_pad = jnp.concatenate(
        [b_lin, jnp.full((128 - C,), -1e30, jnp.float32)]).reshape(1, 128)
    y2d = y.astype(jnp.int32).reshape(N, 1)

    xe_pad = _emb_gather(emb, x_pad)
    xiou, xf2, h2, c2, hf2 = _tc_pre(
        xe_pad, W_iou, b_iou.reshape(1, -1), W_f, b_f.reshape(1, -1), U_f)

    xf_flat = xf2.reshape(2 * N, HH)
    for _ in range(2):
        ht, fc = _edge_phase(h2.reshape(2 * N, HH), c2.reshape(2 * N, HH),
                             hf2.reshape(2 * N, HH), xf_flat,
                             src_plus, dst_plus, dst, zeros_half)
        h2, c2, hf2 = _tc_step(ht.reshape(2, N, HH), fc.reshape(2, N, HH),
                               xiou, U_iou, U_f)

    ht, fc = _edge_phase(h2.reshape(2 * N, HH), c2.reshape(2 * N, HH),
                         hf2.reshape(2 * N, HH), xf_flat,
                         src_plus, dst_plus, dst, zeros_half)
    loss = _tc_final(ht.reshape(2, N, HH), fc.reshape(2, N, HH),
                     xiou, y2d, U_iou, W_lin_pad, b_lin_pad)
    return loss[0, 0]


# speedup vs baseline: 4.2933x; 1.3123x over previous
import functools

import jax
import jax.numpy as jnp
from jax import lax
from jax.experimental import pallas as pl
from jax.experimental.pallas import tpu as pltpu
from jax.experimental.pallas import tpu_sc as plsc

N = 10000
E = 160000
X = 256
H = 256
HH = 128
C = 5

NW = 32
NP = 10240
GCH = 80
EPS = E // 16
EC = 40
NCH = EPS // EC
NP2 = NCH // 2
RPS = 624
RB = 1000
GRID = N // RB


def _sc_mesh():
    return plsc.VectorSubcoreMesh(core_axis_name="c", subcore_axis_name="s")


def _emb_gather_body(emb_hbm, xpad_hbm, out_hbm, idx_v, rows_v, sem):
    cid = lax.axis_index("c")
    sid = lax.axis_index("s")
    wid = sid * 2 + cid
    base = wid * (NP // NW)
    for k in range(NP // NW // GCH):
        off = base + k * GCH
        pltpu.sync_copy(xpad_hbm.at[pl.ds(off, GCH)], idx_v)
        pltpu.async_copy(emb_hbm.at[idx_v], rows_v, sem).wait()
        pltpu.sync_copy(rows_v, out_hbm.at[pl.ds(off, GCH)])


def _emb_gather(emb, x_pad):
    return pl.kernel(
        _emb_gather_body,
        out_type=jax.ShapeDtypeStruct((NP, X), jnp.float32),
        mesh=_sc_mesh(),
        scratch_types=[
            pltpu.VMEM((GCH,), jnp.int32),
            pltpu.VMEM((GCH, X), jnp.float32),
            pltpu.SemaphoreType.DMA,
        ],
    )(emb, x_pad)


def _edge_body(h_hbm, hfc_hbm, xf_hbm, idx3_hbm,
               zeros_hbm, ht_hbm, fc_hbm,
               acc, ib0, ib1,
               hfc0, hfc1, xfb0, xfb1, p0, p1,
               sg0, sg1, ss0, ss1):
    cid = lax.axis_index("c")
    sid = lax.axis_index("s")
    noff = cid * N
    rbase = sid * RPS
    tail = 16 * RPS

    def _zero_acc():
        pltpu.sync_copy(zeros_hbm.at[pl.ds(rbase, RPS)],
                        acc.at[pl.ds(rbase, RPS)])

        @pl.when(sid == 15)
        def _():
            pltpu.sync_copy(zeros_hbm.at[pl.ds(tail, N - tail)],
                            acc.at[pl.ds(tail, N - tail)])

    def _flush_acc(out_hbm):
        pltpu.sync_copy(acc.at[pl.ds(rbase, RPS)],
                        out_hbm.at[pl.ds(noff + rbase, RPS)])

        @pl.when(sid == 15)
        def _():
            pltpu.sync_copy(acc.at[pl.ds(tail, N - tail)],
                            out_hbm.at[pl.ds(noff + tail, N - tail)])

    def _wait_scat(buf, ib, sem):
        pltpu.make_async_copy(buf, acc.at[ib.at[2]], sem).wait()

    def _load_idx(k, ib):
        pltpu.sync_copy(idx3_hbm.at[cid, sid, k], ib)

    _zero_acc()
    plsc.subcore_barrier()

    _load_idx(0, ib0)
    pltpu.async_copy(h_hbm.at[ib0.at[0]], p0, sg0)

    def pass_a(t, carry):
        o = 2 * t + 1

        @pl.when(t > 0)
        def _():
            _wait_scat(p1, ib1, ss1)

        _load_idx(o, ib1)
        pltpu.async_copy(h_hbm.at[ib1.at[0]], p1, sg1)

        pltpu.make_async_copy(h_hbm.at[ib0.at[0]], p0, sg0).wait()
        pltpu.async_copy(p0, acc.at[ib0.at[2]], ss0, add=True)

        @pl.when(t < NP2 - 1)
        def _():
            _wait_scat(p0, ib0, ss0)
            _load_idx(o + 1, ib0)
            pltpu.async_copy(h_hbm.at[ib0.at[0]], p0, sg0)

        pltpu.make_async_copy(h_hbm.at[ib1.at[0]], p1, sg1).wait()
        pltpu.async_copy(p1, acc.at[ib1.at[2]], ss1, add=True)
        return carry

    lax.fori_loop(0, NP2, pass_a, 0)
    _wait_scat(p0, ib0, ss0)
    _wait_scat(p1, ib1, ss1)
    plsc.subcore_barrier()
    _flush_acc(ht_hbm)
    _zero_acc()
    plsc.subcore_barrier()

    def _fireB(ib, hfcb, xfb, sg):
        pltpu.async_copy(hfc_hbm.at[ib.at[0]], hfcb, sg)
        pltpu.async_copy(xf_hbm.at[ib.at[1]], xfb, sg)

    def _waitB(ib, hfcb, xfb, sg):
        pltpu.make_async_copy(hfc_hbm.at[ib.at[0]], hfcb, sg).wait()
        pltpu.make_async_copy(xf_hbm.at[ib.at[1]], xfb, sg).wait()

    def _compute(hfcb, xfb, prod):
        @functools.partial(plsc.parallel_loop, 0, EC, unroll=2)
        def _(r):
            for j in range(HH // 16):
                sl = pl.ds(j * 16, 16)
                z = hfcb[r, pl.ds(j * 16, 16)] + xfb[r, sl]
                prod[r, sl] = hfcb[r, pl.ds(HH + j * 16, 16)] / (1.0 + jnp.exp(-z))

    _load_idx(0, ib0)
    _fireB(ib0, hfc0, xfb0, sg0)

    def pass_b(t, carry):
        o = 2 * t + 1

        @pl.when(t > 0)
        def _():
            _wait_scat(p1, ib1, ss1)

        _load_idx(o, ib1)
        _fireB(ib1, hfc1, xfb1, sg1)

        _waitB(ib0, hfc0, xfb0, sg0)
        _compute(hfc0, xfb0, p0)
        pltpu.async_copy(p0, acc.at[ib0.at[2]], ss0, add=True)

        @pl.when(t < NP2 - 1)
        def _():
            _wait_scat(p0, ib0, ss0)
            _load_idx(o + 1, ib0)
            _fireB(ib0, hfc0, xfb0, sg0)

        _waitB(ib1, hfc1, xfb1, sg1)
        _compute(hfc1, xfb1, p1)
        pltpu.async_copy(p1, acc.at[ib1.at[2]], ss1, add=True)
        return carry

    lax.fori_loop(0, NP2, pass_b, 0)
    _wait_scat(p0, ib0, ss0)
    _wait_scat(p1, ib1, ss1)
    plsc.subcore_barrier()
    _flush_acc(fc_hbm)


def _edge_phase(h2, hfc2, xf2, idx3, zeros_half):
    buf = lambda w: pltpu.VMEM((EC, w), jnp.float32)
    return pl.kernel(
        _edge_body,
        out_type=(jax.ShapeDtypeStruct((2 * N, HH), jnp.float32),
                  jax.ShapeDtypeStruct((2 * N, HH), jnp.float32)),
        mesh=_sc_mesh(),
        scratch_types=[
            pltpu.VMEM_SHARED((N, HH), jnp.float32),
            pltpu.VMEM((3, EC), jnp.int32), pltpu.VMEM((3, EC), jnp.int32),
            buf(H), buf(H), buf(HH), buf(HH), buf(HH), buf(HH),
            pltpu.SemaphoreType.DMA, pltpu.SemaphoreType.DMA,
            pltpu.SemaphoreType.DMA, pltpu.SemaphoreType.DMA,
        ],
    )(h2, hfc2, xf2, idx3, zeros_half)


def _split_store(ref, full):
    ref[0] = full[:, :HH]
    ref[1] = full[:, HH:]


def _hfc_store(ref, hf, c):
    ref[0, :, :HH] = hf[:, :HH]
    ref[0, :, HH:] = c[:, :HH]
    ref[1, :, :HH] = hf[:, HH:]
    ref[1, :, HH:] = c[:, HH:]


def _pre_body(xe_ref, wiou_ref, biou_ref, wf_ref, bf_ref, uf_ref,
              xiou_ref, xf_ref, h0_ref, hfc1_ref):
    xe = xe_ref[...]
    xiou = jnp.dot(xe, wiou_ref[...]) + biou_ref[...]
    xiou_ref[...] = xiou
    xf = jnp.dot(xe, wf_ref[...]) + bf_ref[...]
    _split_store(xf_ref, xf)
    i = jax.nn.sigmoid(xiou[:, :H])
    o = jax.nn.sigmoid(xiou[:, H:2 * H])
    u = jnp.tanh(xiou[:, 2 * H:])
    c0 = i * u
    h0 = o * jnp.tanh(c0)
    _split_store(h0_ref, h0)
    _hfc_store(hfc1_ref, jnp.dot(h0, uf_ref[...]), c0)


def _tc_pre(xe_pad, W_iou, b_iou, W_f, b_f, U_f):
    out3 = jax.ShapeDtypeStruct((2, N, HH), jnp.float32)
    out6 = jax.ShapeDtypeStruct((2, N, H), jnp.float32)
    full = lambda *_: (0, 0)
    split3 = pl.BlockSpec((2, RB, HH), lambda i: (0, i, 0))
    split6 = pl.BlockSpec((2, RB, H), lambda i: (0, i, 0))
    return pl.pallas_call(
        _pre_body,
        grid=(GRID,),
        in_specs=[
            pl.BlockSpec((RB, X), lambda i: (i, 0)),
            pl.BlockSpec((X, 3 * H), full),
            pl.BlockSpec((1, 3 * H), full),
            pl.BlockSpec((X, H), full),
            pl.BlockSpec((1, H), full),
            pl.BlockSpec((H, H), full),
        ],
        out_specs=[
            pl.BlockSpec((RB, 3 * H), lambda i: (i, 0)),
            split3, split3, split6,
        ],
        out_shape=[jax.ShapeDtypeStruct((N, 3 * H), jnp.float32),
                   out3, out3, out6],
    )(xe_pad, W_iou, b_iou, W_f, b_f, U_f)


def _gate_update(ht_ref, fc_ref, xiou_ref, uiou_ref):
    htil = jnp.concatenate([ht_ref[0], ht_ref[1]], axis=1)
    iou = xiou_ref[...] + jnp.dot(htil, uiou_ref[...])
    i = jax.nn.sigmoid(iou[:, :H])
    o = jax.nn.sigmoid(iou[:, H:2 * H])
    u = jnp.tanh(iou[:, 2 * H:])
    c = i * u + jnp.concatenate([fc_ref[0], fc_ref[1]], axis=1)
    h = o * jnp.tanh(c)
    return h, c


def _step_body(ht_ref, fc_ref, xiou_ref, uiou_ref, uf_ref,
               h_ref, hfc_ref):
    h, c = _gate_update(ht_ref, fc_ref, xiou_ref, uiou_ref)
    _split_store(h_ref, h)
    _hfc_store(hfc_ref, jnp.dot(h, uf_ref[...]), c)


def _tc_step(ht2, fc2, xiou, U_iou, U_f):
    out3 = jax.ShapeDtypeStruct((2, N, HH), jnp.float32)
    out6 = jax.ShapeDtypeStruct((2, N, H), jnp.float32)
    full = lambda *_: (0, 0)
    split3 = pl.BlockSpec((2, RB, HH), lambda i: (0, i, 0))
    split6 = pl.BlockSpec((2, RB, H), lambda i: (0, i, 0))
    return pl.pallas_call(
        _step_body,
        grid=(GRID,),
        in_specs=[
            split3, split3,
            pl.BlockSpec((RB, 3 * H), lambda i: (i, 0)),
            pl.BlockSpec((H, 3 * H), full),
            pl.BlockSpec((H, H), full),
        ],
        out_specs=[split3, split6],
        out_shape=[out3, out6],
    )(ht2, fc2, xiou, U_iou, U_f)


def _final_body(ht_ref, fc_ref, xiou_ref, y_ref, uiou_ref, wlin_ref, blin_ref,
                loss_ref):
    h, _ = _gate_update(ht_ref, fc_ref, xiou_ref, uiou_ref)
    logits = jnp.dot(h, wlin_ref[...]) + blin_ref[...]
    m = jnp.max(logits, axis=1, keepdims=True)
    lse = m + jnp.log(jnp.sum(jnp.exp(logits - m), axis=1, keepdims=True))
    y = y_ref[...]
    sel = lax.broadcasted_iota(jnp.int32, (RB, 128), 1) == y
    ly = jnp.sum(jnp.where(sel, logits, 0.0), axis=1, keepdims=True)
    part = jnp.sum(lse - ly, keepdims=True).reshape(1, 1) * (1.0 / N)

    @pl.when(pl.program_id(0) == 0)
    def _():
        loss_ref[...] = jnp.zeros((1, 1), jnp.float32)

    loss_ref[...] += part


def _tc_final(ht2, fc2, xiou, y2d, U_iou, W_lin_pad, b_lin_pad):
    full = lambda *_: (0, 0)
    split3 = pl.BlockSpec((2, RB, HH), lambda i: (0, i, 0))
    return pl.pallas_call(
        _final_body,
        grid=(GRID,),
        in_specs=[
            split3, split3,
            pl.BlockSpec((RB, 3 * H), lambda i: (i, 0)),
            pl.BlockSpec((RB, 1), lambda i: (i, 0)),
            pl.BlockSpec((H, 3 * H), full),
            pl.BlockSpec((H, 128), full),
            pl.BlockSpec((1, 128), full),
        ],
        out_specs=pl.BlockSpec((1, 1), full),
        out_shape=jax.ShapeDtypeStruct((1, 1), jnp.float32),
    )(ht2, fc2, xiou, y2d, U_iou, W_lin_pad, b_lin_pad)


def kernel(x, edge_index, y, emb, W_iou, U_iou, b_iou, W_f, U_f, b_f,
           W_lin, b_lin):
    src = edge_index[0].astype(jnp.int32)
    dst = edge_index[1].astype(jnp.int32)
    offs = jnp.array([0, N], jnp.int32).reshape(2, 1, 1, 1)
    src_r = src.reshape(1, 16, NCH, EC)
    dst_r = dst.reshape(1, 16, NCH, EC)
    idx3 = jnp.stack([src_r + offs, dst_r + offs,
                      jnp.broadcast_to(dst_r, (2, 16, NCH, EC))], axis=3)
    x_pad = jnp.concatenate([x.astype(jnp.int32),
                             jnp.zeros((NP - N,), jnp.int32)])
    zeros_half = jnp.zeros((N, HH), jnp.float32)
    W_lin_pad = jnp.concatenate(
        [W_lin, jnp.zeros((H, 128 - C), jnp.float32)], axis=1)
    b_lin_pad = jnp.concatenate(
        [b_lin, jnp.full((128 - C,), -1e30, jnp.float32)]).reshape(1, 128)
    y2d = y.astype(jnp.int32).reshape(N, 1)

    xe_pad = _emb_gather(emb, x_pad)
    xiou, xf2, h2, hfc2 = _tc_pre(
        xe_pad, W_iou, b_iou.reshape(1, -1), W_f, b_f.reshape(1, -1), U_f)

    xf_flat = xf2.reshape(2 * N, HH)
    for _ in range(2):
        ht, fc = _edge_phase(h2.reshape(2 * N, HH), hfc2.reshape(2 * N, H),
                             xf_flat, idx3, zeros_half)
        h2, hfc2 = _tc_step(ht.reshape(2, N, HH), fc.reshape(2, N, HH),
                            xiou, U_iou, U_f)

    ht, fc = _edge_phase(h2.reshape(2 * N, HH), hfc2.reshape(2 * N, H),
                         xf_flat, idx3, zeros_half)
    loss = _tc_final(ht.reshape(2, N, HH), fc.reshape(2, N, HH),
                     xiou, y2d, U_iou, W_lin_pad, b_lin_pad)
    return loss[0, 0]
